# baseline (device time: 710231 ns/iter reference)
import jax
import jax.numpy as jnp
from jax import lax
from jax.experimental import pallas as pl
from jax.experimental.pallas import tpu as pltpu

N_DEV = 8
N_TOK = 16384
M_TOK = N_TOK // N_DEV
D_IN = 512
D_OUT = 1024
N_EXP = 64
E_LOC = N_EXP // N_DEV
CAP = 204
CAP_PAD = 208
BLK = E_LOC * CAP_PAD


def _neighbor_barrier(my_pos):
    left = lax.rem(my_pos - 1 + N_DEV, N_DEV)
    right = lax.rem(my_pos + 1, N_DEV)
    barrier_sem = pltpu.get_barrier_semaphore()
    for nbr in (left, right):
        pl.semaphore_signal(
            barrier_sem, inc=1,
            device_id=(nbr,), device_id_type=pl.DeviceIdType.MESH,
        )
    pl.semaphore_wait(barrier_sem, 2)
    return right


def _ring_hops(ref, m_per, my_pos, right, send_sems, recv_sems):
    for h in range(N_DEV - 1):
        send_origin = lax.rem(my_pos - h + N_DEV, N_DEV)
        rdma = pltpu.make_async_remote_copy(
            src_ref=ref.at[pl.ds(send_origin * m_per, m_per), :],
            dst_ref=ref.at[pl.ds(send_origin * m_per, m_per), :],
            send_sem=send_sems.at[h],
            recv_sem=recv_sems.at[h],
            device_id=(right,),
            device_id_type=pl.DeviceIdType.MESH,
        )
        rdma.start()
        rdma.wait()


def _ring_allgather(x_shard, collective_id):
    m_per, n = x_shard.shape

    def body(x_ref, out_ref, send_sems, recv_sems):
        my_pos = lax.axis_index("i")
        right = _neighbor_barrier(my_pos)
        out_ref[pl.ds(my_pos * m_per, m_per), :] = x_ref[...]
        _ring_hops(out_ref, m_per, my_pos, right, send_sems, recv_sems)

    return pl.pallas_call(
        body,
        out_shape=jax.ShapeDtypeStruct((N_DEV * m_per, n), x_shard.dtype),
        in_specs=[pl.BlockSpec(memory_space=pltpu.VMEM)],
        out_specs=pl.BlockSpec(memory_space=pltpu.VMEM),
        scratch_shapes=[
            pltpu.SemaphoreType.DMA((N_DEV - 1,)),
            pltpu.SemaphoreType.DMA((N_DEV - 1,)),
        ],
        compiler_params=pltpu.CompilerParams(collective_id=collective_id),
    )(x_shard)


def _matmul_table_ag(compact_x, expert_W_bf):

    def body(cx_ref, w_ref, table_ref, send_sems, recv_sems):
        my_pos = lax.axis_index("i")
        right = _neighbor_barrier(my_pos)
        for j in range(E_LOC):
            res = jnp.dot(cx_ref[j], w_ref[j],
                          preferred_element_type=jnp.float32)
            base = (my_pos * E_LOC + j) * CAP_PAD
            table_ref[pl.ds(base, CAP_PAD), :] = res.astype(jnp.bfloat16)
        _ring_hops(table_ref, BLK, my_pos, right, send_sems, recv_sems)

    return pl.pallas_call(
        body,
        out_shape=jax.ShapeDtypeStruct((N_DEV * BLK, D_OUT), jnp.bfloat16),
        in_specs=[
            pl.BlockSpec(memory_space=pltpu.VMEM),
            pl.BlockSpec(memory_space=pltpu.VMEM),
        ],
        out_specs=pl.BlockSpec(memory_space=pltpu.VMEM),
        scratch_shapes=[
            pltpu.SemaphoreType.DMA((N_DEV - 1,)),
            pltpu.SemaphoreType.DMA((N_DEV - 1,)),
        ],
        compiler_params=pltpu.CompilerParams(collective_id=2),
    )(compact_x, expert_W_bf)


_PIPE = 16


def _rows_gather(src3, idx):
    n_rows = idx.shape[0]
    _, a, b = src3.shape
    assert n_rows % _PIPE == 0

    def body(src_ref, idx_ref, out_ref, sems):
        def make(s, k):
            r = idx_ref[s]
            return pltpu.make_async_copy(
                src_ref.at[r], out_ref.at[s], sems.at[k])

        n_batch = n_rows // _PIPE

        def outer(bi, _):
            for k in range(_PIPE):
                s = bi * _PIPE + k

                @pl.when(bi > 0)
                def _():
                    make(s - _PIPE, k).wait()

                make(s, k).start()
            return 0

        lax.fori_loop(0, n_batch, outer, 0)
        for k in range(_PIPE):
            make((n_batch - 1) * _PIPE + k, k).wait()

    return pl.pallas_call(
        body,
        out_shape=jax.ShapeDtypeStruct((n_rows, a, b), src3.dtype),
        in_specs=[
            pl.BlockSpec(memory_space=pltpu.HBM),
            pl.BlockSpec(memory_space=pltpu.SMEM),
        ],
        out_specs=pl.BlockSpec(memory_space=pltpu.VMEM),
        scratch_shapes=[pltpu.SemaphoreType.DMA((_PIPE,))],
    )(src3, idx)


def kernel(x, router_W, route_idx, expert_W):
    del router_W
    my_pos = lax.axis_index("i")

    route_2d = route_idx.reshape(M_TOK // 128, 128)
    route_all = _ring_allgather(route_2d, collective_id=0).reshape(N_TOK)

    onehot = (route_all[:, None] == jnp.arange(N_EXP)[None, :]).astype(jnp.int32)
    csum = jnp.cumsum(onehot, axis=0)
    rank = (csum * onehot).sum(axis=1)
    kept_all = rank <= CAP
    slot_all = rank - 1

    tok_ids = jnp.arange(N_TOK, dtype=jnp.int32)
    my_exp = my_pos * E_LOC + jnp.arange(E_LOC, dtype=jnp.int32)
    mine = kept_all[None, :] & (route_all[None, :] == my_exp[:, None])
    order = jnp.where(mine, tok_ids[None, :], jnp.int32(N_TOK))
    idx = jnp.sort(order, axis=1)[:, :CAP_PAD]
    safe_idx = jnp.minimum(idx, N_TOK - 1)

    lo = my_pos * M_TOK
    e_loc = lax.dynamic_slice(route_all, (lo,), (M_TOK,))
    s_loc = lax.dynamic_slice(slot_all, (lo,), (M_TOK,))
    k_loc = lax.dynamic_slice(kept_all, (lo,), (M_TOK,))
    row_meta = jnp.where(k_loc, e_loc * CAP_PAD + s_loc, 0).astype(jnp.int32)

    x_all = _ring_allgather(x.astype(jnp.bfloat16), collective_id=1)

    compact_x = _rows_gather(
        x_all.reshape(N_TOK, 4, 128), safe_idx.reshape(BLK)
    ).reshape(E_LOC, CAP_PAD, D_IN)
    table = _matmul_table_ag(compact_x, expert_W.astype(jnp.bfloat16))

    staged = _rows_gather(table.reshape(N_DEV * BLK, 8, 128), row_meta)
    rows = staged.reshape(M_TOK, D_OUT).astype(jnp.float32)
    return jnp.where(k_loc[:, None], rows, jnp.float32(0))


# device time: 521892 ns/iter; 1.3609x vs baseline; 1.3609x over previous
import jax
import jax.numpy as jnp
from jax import lax
from jax.experimental import pallas as pl
from jax.experimental.pallas import tpu as pltpu

N_DEV = 8
N_TOK = 16384
M_TOK = N_TOK // N_DEV
D_IN = 512
D_OUT = 1024
N_EXP = 64
E_LOC = N_EXP // N_DEV
CAP = 204
CAP_PAD = 208
BLK = E_LOC * CAP_PAD


def _neighbor_barrier(my_pos):
    left = lax.rem(my_pos - 1 + N_DEV, N_DEV)
    right = lax.rem(my_pos + 1, N_DEV)
    barrier_sem = pltpu.get_barrier_semaphore()
    for nbr in (left, right):
        pl.semaphore_signal(
            barrier_sem, inc=1,
            device_id=(nbr,), device_id_type=pl.DeviceIdType.MESH,
        )
    pl.semaphore_wait(barrier_sem, 2)
    return left, right


_CW = N_DEV // 2
_CCW = N_DEV - 1 - _CW


def _ring_hops(ref, m_per, my_pos, left, right, cw_s, cw_r, ccw_s, ccw_r):

    def _hop(origin, sems_s, sems_r, h, dst):
        return pltpu.make_async_remote_copy(
            src_ref=ref.at[pl.ds(origin * m_per, m_per), :],
            dst_ref=ref.at[pl.ds(origin * m_per, m_per), :],
            send_sem=sems_s.at[h],
            recv_sem=sems_r.at[h],
            device_id=(dst,),
            device_id_type=pl.DeviceIdType.MESH,
        )

    for h in range(_CW):
        o_cw = lax.rem(my_pos - h + N_DEV, N_DEV)
        cw = _hop(o_cw, cw_s, cw_r, h, right)
        cw.start()
        ccw = None
        if h < _CCW:
            o_ccw = lax.rem(my_pos + h, N_DEV)
            ccw = _hop(o_ccw, ccw_s, ccw_r, h, left)
            ccw.start()
        cw.wait()
        if ccw is not None:
            ccw.wait()


def _ring_allgather(x_shard, collective_id):
    m_per, n = x_shard.shape

    def body(x_ref, out_ref, cw_s, cw_r, ccw_s, ccw_r):
        my_pos = lax.axis_index("i")
        left, right = _neighbor_barrier(my_pos)
        out_ref[pl.ds(my_pos * m_per, m_per), :] = x_ref[...]
        _ring_hops(out_ref, m_per, my_pos, left, right,
                   cw_s, cw_r, ccw_s, ccw_r)

    return pl.pallas_call(
        body,
        out_shape=jax.ShapeDtypeStruct((N_DEV * m_per, n), x_shard.dtype),
        in_specs=[pl.BlockSpec(memory_space=pltpu.VMEM)],
        out_specs=pl.BlockSpec(memory_space=pltpu.VMEM),
        scratch_shapes=[
            pltpu.SemaphoreType.DMA((_CW,)),
            pltpu.SemaphoreType.DMA((_CW,)),
            pltpu.SemaphoreType.DMA((_CCW,)),
            pltpu.SemaphoreType.DMA((_CCW,)),
        ],
        compiler_params=pltpu.CompilerParams(collective_id=collective_id),
    )(x_shard)


def _matmul_table_ag(compact_x, expert_W_bf):

    def body(cx_ref, w_ref, table_ref, cw_s, cw_r, ccw_s, ccw_r):
        my_pos = lax.axis_index("i")
        left, right = _neighbor_barrier(my_pos)
        for j in range(E_LOC):
            res = jnp.dot(cx_ref[j], w_ref[j],
                          preferred_element_type=jnp.float32)
            base = (my_pos * E_LOC + j) * CAP_PAD
            table_ref[pl.ds(base, CAP_PAD), :] = res.astype(jnp.bfloat16)
        _ring_hops(table_ref, BLK, my_pos, left, right,
                   cw_s, cw_r, ccw_s, ccw_r)

    return pl.pallas_call(
        body,
        out_shape=jax.ShapeDtypeStruct((N_DEV * BLK, D_OUT), jnp.bfloat16),
        in_specs=[
            pl.BlockSpec(memory_space=pltpu.VMEM),
            pl.BlockSpec(memory_space=pltpu.VMEM),
        ],
        out_specs=pl.BlockSpec(memory_space=pltpu.VMEM),
        scratch_shapes=[
            pltpu.SemaphoreType.DMA((_CW,)),
            pltpu.SemaphoreType.DMA((_CW,)),
            pltpu.SemaphoreType.DMA((_CCW,)),
            pltpu.SemaphoreType.DMA((_CCW,)),
        ],
        compiler_params=pltpu.CompilerParams(collective_id=2),
    )(compact_x, expert_W_bf)


_PIPE = 16


def _rows_gather(src3, idx):
    n_rows = idx.shape[0]
    _, a, b = src3.shape
    assert n_rows % _PIPE == 0

    def body(src_ref, idx_ref, out_ref, sems):
        def make(s, k):
            r = idx_ref[s]
            return pltpu.make_async_copy(
                src_ref.at[r], out_ref.at[s], sems.at[k])

        n_batch = n_rows // _PIPE

        def outer(bi, _):
            for k in range(_PIPE):
                s = bi * _PIPE + k

                @pl.when(bi > 0)
                def _():
                    make(s - _PIPE, k).wait()

                make(s, k).start()
            return 0

        lax.fori_loop(0, n_batch, outer, 0)
        for k in range(_PIPE):
            make((n_batch - 1) * _PIPE + k, k).wait()

    return pl.pallas_call(
        body,
        out_shape=jax.ShapeDtypeStruct((n_rows, a, b), src3.dtype),
        in_specs=[
            pl.BlockSpec(memory_space=pltpu.HBM),
            pl.BlockSpec(memory_space=pltpu.SMEM),
        ],
        out_specs=pl.BlockSpec(memory_space=pltpu.VMEM),
        scratch_shapes=[pltpu.SemaphoreType.DMA((_PIPE,))],
    )(src3, idx)


def kernel(x, router_W, route_idx, expert_W):
    del router_W
    my_pos = lax.axis_index("i")

    route_2d = route_idx.reshape(M_TOK // 128, 128)
    route_all = _ring_allgather(route_2d, collective_id=0).reshape(N_TOK)

    onehot = (route_all[:, None] == jnp.arange(N_EXP)[None, :]).astype(jnp.int32)
    csum = jnp.cumsum(onehot, axis=0)
    rank = (csum * onehot).sum(axis=1)
    kept_all = rank <= CAP
    slot_all = rank - 1

    tok_ids = jnp.arange(N_TOK, dtype=jnp.int32)
    my_exp = my_pos * E_LOC + jnp.arange(E_LOC, dtype=jnp.int32)
    mine = kept_all[None, :] & (route_all[None, :] == my_exp[:, None])
    order = jnp.where(mine, tok_ids[None, :], jnp.int32(N_TOK))
    idx = jnp.sort(order, axis=1)[:, :CAP_PAD]
    safe_idx = jnp.minimum(idx, N_TOK - 1)

    lo = my_pos * M_TOK
    e_loc = lax.dynamic_slice(route_all, (lo,), (M_TOK,))
    s_loc = lax.dynamic_slice(slot_all, (lo,), (M_TOK,))
    k_loc = lax.dynamic_slice(kept_all, (lo,), (M_TOK,))
    row_meta = jnp.where(k_loc, e_loc * CAP_PAD + s_loc, 0).astype(jnp.int32)

    x_all = _ring_allgather(x.astype(jnp.bfloat16), collective_id=1)

    compact_x = _rows_gather(
        x_all.reshape(N_TOK, 4, 128), safe_idx.reshape(BLK)
    ).reshape(E_LOC, CAP_PAD, D_IN)
    table = _matmul_table_ag(compact_x, expert_W.astype(jnp.bfloat16))

    staged = _rows_gather(table.reshape(N_DEV * BLK, 8, 128), row_meta)
    rows = staged.reshape(M_TOK, D_OUT).astype(jnp.float32)
    return jnp.where(k_loc[:, None], rows, jnp.float32(0))


# device time: 455287 ns/iter; 1.5600x vs baseline; 1.1463x over previous
import jax
import jax.numpy as jnp
from jax import lax
from jax.experimental import pallas as pl
from jax.experimental.pallas import tpu as pltpu

N_DEV = 8
N_TOK = 16384
M_TOK = N_TOK // N_DEV
D_IN = 512
D_OUT = 1024
N_EXP = 64
E_LOC = N_EXP // N_DEV
CAP = 204
CAP_PAD = 208
BLK = E_LOC * CAP_PAD


def _neighbor_barrier(my_pos):
    left = lax.rem(my_pos - 1 + N_DEV, N_DEV)
    right = lax.rem(my_pos + 1, N_DEV)
    barrier_sem = pltpu.get_barrier_semaphore()
    for nbr in (left, right):
        pl.semaphore_signal(
            barrier_sem, inc=1,
            device_id=(nbr,), device_id_type=pl.DeviceIdType.MESH,
        )
    pl.semaphore_wait(barrier_sem, 2)
    return left, right


_CW = N_DEV // 2
_CCW = N_DEV - 1 - _CW


def _ring_hops(ref, m_per, my_pos, left, right, cw_s, cw_r, ccw_s, ccw_r):

    def _hop(origin, sems_s, sems_r, h, dst):
        return pltpu.make_async_remote_copy(
            src_ref=ref.at[pl.ds(origin * m_per, m_per), :],
            dst_ref=ref.at[pl.ds(origin * m_per, m_per), :],
            send_sem=sems_s.at[h],
            recv_sem=sems_r.at[h],
            device_id=(dst,),
            device_id_type=pl.DeviceIdType.MESH,
        )

    for h in range(_CW):
        o_cw = lax.rem(my_pos - h + N_DEV, N_DEV)
        cw = _hop(o_cw, cw_s, cw_r, h, right)
        cw.start()
        ccw = None
        if h < _CCW:
            o_ccw = lax.rem(my_pos + h, N_DEV)
            ccw = _hop(o_ccw, ccw_s, ccw_r, h, left)
            ccw.start()
        cw.wait()
        if ccw is not None:
            ccw.wait()


def _ring_allgather(x_shard, collective_id):
    m_per, n = x_shard.shape

    def body(x_ref, out_ref, cw_s, cw_r, ccw_s, ccw_r):
        my_pos = lax.axis_index("i")
        left, right = _neighbor_barrier(my_pos)
        out_ref[pl.ds(my_pos * m_per, m_per), :] = x_ref[...]
        _ring_hops(out_ref, m_per, my_pos, left, right,
                   cw_s, cw_r, ccw_s, ccw_r)

    return pl.pallas_call(
        body,
        out_shape=jax.ShapeDtypeStruct((N_DEV * m_per, n), x_shard.dtype),
        in_specs=[pl.BlockSpec(memory_space=pltpu.VMEM)],
        out_specs=pl.BlockSpec(memory_space=pltpu.VMEM),
        scratch_shapes=[
            pltpu.SemaphoreType.DMA((_CW,)),
            pltpu.SemaphoreType.DMA((_CW,)),
            pltpu.SemaphoreType.DMA((_CCW,)),
            pltpu.SemaphoreType.DMA((_CCW,)),
        ],
        compiler_params=pltpu.CompilerParams(collective_id=collective_id),
    )(x_shard)


def _matmul_table_ag(compact_x, expert_W_bf):

    def body(cx_ref, w_ref, table_ref, cw_s, cw_r, ccw_s, ccw_r):
        my_pos = lax.axis_index("i")
        left, right = _neighbor_barrier(my_pos)
        for j in range(E_LOC):
            res = jnp.dot(cx_ref[j], w_ref[j],
                          preferred_element_type=jnp.float32)
            base = (my_pos * E_LOC + j) * CAP_PAD
            table_ref[pl.ds(base, CAP_PAD), :] = res.astype(jnp.bfloat16)
        _ring_hops(table_ref, BLK, my_pos, left, right,
                   cw_s, cw_r, ccw_s, ccw_r)

    return pl.pallas_call(
        body,
        out_shape=jax.ShapeDtypeStruct((N_DEV * BLK, D_OUT), jnp.bfloat16),
        in_specs=[
            pl.BlockSpec(memory_space=pltpu.VMEM),
            pl.BlockSpec(memory_space=pltpu.VMEM),
        ],
        out_specs=pl.BlockSpec(memory_space=pltpu.VMEM),
        scratch_shapes=[
            pltpu.SemaphoreType.DMA((_CW,)),
            pltpu.SemaphoreType.DMA((_CW,)),
            pltpu.SemaphoreType.DMA((_CCW,)),
            pltpu.SemaphoreType.DMA((_CCW,)),
        ],
        compiler_params=pltpu.CompilerParams(collective_id=2),
    )(compact_x, expert_W_bf)


_PIPE = 64


def _rows_gather(src3, idx):
    n_rows = idx.shape[0]
    _, a, b = src3.shape
    assert n_rows % _PIPE == 0

    def body(src_ref, idx_ref, out_ref, sems):
        def make(s, k):
            r = idx_ref[s]
            return pltpu.make_async_copy(
                src_ref.at[r], out_ref.at[s], sems.at[k])

        n_batch = n_rows // _PIPE

        def outer(bi, _):
            for k in range(_PIPE):
                s = bi * _PIPE + k

                @pl.when(bi > 0)
                def _():
                    make(s - _PIPE, k).wait()

                make(s, k).start()
            return 0

        lax.fori_loop(0, n_batch, outer, 0)
        for k in range(_PIPE):
            make((n_batch - 1) * _PIPE + k, k).wait()

    return pl.pallas_call(
        body,
        out_shape=jax.ShapeDtypeStruct((n_rows, a, b), src3.dtype),
        in_specs=[
            pl.BlockSpec(memory_space=pltpu.HBM),
            pl.BlockSpec(memory_space=pltpu.SMEM),
        ],
        out_specs=pl.BlockSpec(memory_space=pltpu.VMEM),
        scratch_shapes=[pltpu.SemaphoreType.DMA((_PIPE,))],
    )(src3, idx)


def kernel(x, router_W, route_idx, expert_W):
    del router_W
    my_pos = lax.axis_index("i")

    route_2d = route_idx.reshape(M_TOK // 128, 128)
    route_all = _ring_allgather(route_2d, collective_id=0).reshape(N_TOK)

    onehot = (route_all[:, None] == jnp.arange(N_EXP)[None, :]).astype(jnp.int32)
    csum = jnp.cumsum(onehot, axis=0)
    rank = (csum * onehot).sum(axis=1)
    kept_all = rank <= CAP
    slot_all = rank - 1

    tok_ids = jnp.arange(N_TOK, dtype=jnp.int32)
    my_exp = my_pos * E_LOC + jnp.arange(E_LOC, dtype=jnp.int32)
    mine = kept_all[None, :] & (route_all[None, :] == my_exp[:, None])
    order = jnp.where(mine, tok_ids[None, :], jnp.int32(N_TOK))
    idx = jnp.sort(order, axis=1)[:, :CAP_PAD]
    safe_idx = jnp.minimum(idx, N_TOK - 1)

    lo = my_pos * M_TOK
    e_loc = lax.dynamic_slice(route_all, (lo,), (M_TOK,))
    s_loc = lax.dynamic_slice(slot_all, (lo,), (M_TOK,))
    k_loc = lax.dynamic_slice(kept_all, (lo,), (M_TOK,))
    row_meta = jnp.where(k_loc, e_loc * CAP_PAD + s_loc, 0).astype(jnp.int32)

    x_all = _ring_allgather(x.astype(jnp.bfloat16), collective_id=1)

    compact_x = _rows_gather(
        x_all.reshape(N_TOK, 4, 128), safe_idx.reshape(BLK)
    ).reshape(E_LOC, CAP_PAD, D_IN)
    table = _matmul_table_ag(compact_x, expert_W.astype(jnp.bfloat16))

    staged = _rows_gather(table.reshape(N_DEV * BLK, 8, 128), row_meta)
    rows = staged.reshape(M_TOK, D_OUT).astype(jnp.float32)
    return jnp.where(k_loc[:, None], rows, jnp.float32(0))
